# baseline (speedup 1.0000x reference)
import functools

import jax
import jax.numpy as jnp
from jax import lax
from jax.experimental import pallas as pl
from jax.experimental.pallas import tpu as pltpu
from jax.experimental.pallas import tpu_sc as plsc

VOCAB = 100000
EMB = 300
NUM_CLASSES = 20
BATCH = 4096
TIME = 200

C_PAD = 32
NC, NS = 2, 16
NW = NC * NS
B_PER_W = BATCH // NW
T_HALF = TIME // 2
T_ROW = 104


_VBLK = 2048


def _proj_body(t_ref, w_ref, b_ref, out_ref):
    out_ref[:, 0:C_PAD] = lax.dot_general(
        t_ref[...], w_ref[...],
        (((0,), (1,)), ((), ())),
        preferred_element_type=jnp.float32,
    ) + b_ref[...]


def _project_table(emb_t, fc_w_pad, fc_b_pad):
    return pl.pallas_call(
        _proj_body,
        grid=((VOCAB + _VBLK - 1) // _VBLK,),
        in_specs=[
            pl.BlockSpec((EMB, _VBLK), lambda i: (0, i)),
            pl.BlockSpec((C_PAD, EMB), lambda i: (0, 0)),
            pl.BlockSpec((1, C_PAD), lambda i: (0, 0)),
        ],
        out_specs=pl.BlockSpec((_VBLK, 128), lambda i: (i, 0)),
        out_shape=jax.ShapeDtypeStruct((VOCAB, 128), jnp.float32),
    )(emb_t, fc_w_pad, fc_b_pad)


T_CHUNK = 8
N_CHUNKS = TIME // T_CHUNK


def _sc_body(xt_hbm, proj_hbm, out_hbm, raw_v, rows_v, out_v, sem0, sem1):
    wid = lax.axis_index("s") * NC + lax.axis_index("c")
    base = wid * B_PER_W
    sems = (sem0, sem1)

    pltpu.sync_copy(xt_hbm.at[:, pl.ds(base, B_PER_W)], raw_v)

    def fire(c, buf):
        for k in range(T_CHUNK):
            pltpu.async_copy(proj_hbm.at[raw_v.at[c * T_CHUNK + k]],
                             rows_v.at[buf, k], sems[buf])

    def drain(c, buf):
        for k in range(T_CHUNK):
            pltpu.make_async_copy(proj_hbm.at[raw_v.at[c * T_CHUNK + k]],
                                  rows_v.at[buf, k], sems[buf]).wait()

    def reduce(buf, first):
        def accum(b, _):
            if first:
                a0 = rows_v[buf, 0, b, 0:16]
                a1 = rows_v[buf, 0, b, 16:32]
                k0 = 1
            else:
                a0 = out_v[b, 0:16]
                a1 = out_v[b, 16:32]
                k0 = 0
            for k in range(k0, T_CHUNK):
                a0 = a0 + rows_v[buf, k, b, 0:16]
                a1 = a1 + rows_v[buf, k, b, 16:32]
            out_v[b, 0:16] = a0
            out_v[b, 16:32] = a1
            return 0

        lax.fori_loop(0, B_PER_W, accum, 0, unroll=4)

    fire(0, 0)
    fire(1, 1)
    drain(0, 0)
    reduce(0, True)

    def step_even(g, _):
        fire(2 * g + 2, 0)
        drain(2 * g + 1, 1)
        reduce(1, False)
        fire(2 * g + 3, 1)
        drain(2 * g + 2, 0)
        reduce(0, False)
        return 0

    lax.fori_loop(0, (N_CHUNKS - 2) // 2, step_even, 0)
    fire(N_CHUNKS - 1, 0)
    drain(N_CHUNKS - 2, 1)
    reduce(1, False)
    drain(N_CHUNKS - 1, 0)
    reduce(0, False)

    pltpu.sync_copy(out_v, out_hbm.at[pl.ds(base, B_PER_W)])


@functools.cache
def _make_sc_kernel():
    return pl.kernel(
        _sc_body,
        out_type=jax.ShapeDtypeStruct((BATCH, C_PAD), jnp.float32),
        mesh=plsc.VectorSubcoreMesh(core_axis_name="c", subcore_axis_name="s"),
        scratch_types=[
            pltpu.VMEM((TIME, B_PER_W), jnp.int32),
            pltpu.VMEM((2, T_CHUNK, B_PER_W, C_PAD), jnp.float32),
            pltpu.VMEM((B_PER_W, C_PAD), jnp.float32),
            pltpu.SemaphoreType.DMA,
            pltpu.SemaphoreType.DMA,
        ],
        compiler_params=pltpu.CompilerParams(use_tc_tiling_on_sc=False,
                                             needs_layout_passes=False),
    )


def kernel(x, emb_table, fc_w, fc_b):
    fc_w_pad = jnp.zeros((C_PAD, EMB), jnp.float32).at[:NUM_CLASSES].set(fc_w)
    fc_w_pad = fc_w_pad * (1.0 / TIME)
    fc_b_pad = jnp.zeros((1, C_PAD), jnp.float32).at[0, :NUM_CLASSES].set(
        fc_b * (1.0 / TIME))
    proj = _project_table(emb_table.T, fc_w_pad, fc_b_pad)
    proj4 = proj.reshape(4 * VOCAB, C_PAD)
    out = _make_sc_kernel()(x.T * 4, proj4)
    return out[:, :NUM_CLASSES]


# speedup vs baseline: 1.1138x; 1.1138x over previous
import functools

import jax
import jax.numpy as jnp
from jax import lax
from jax.experimental import pallas as pl
from jax.experimental.pallas import tpu as pltpu
from jax.experimental.pallas import tpu_sc as plsc

VOCAB = 100000
EMB = 300
NUM_CLASSES = 20
BATCH = 4096
TIME = 200

C_PAD = 32
NC, NS = 2, 16
NW = NC * NS
B_PER_W = BATCH // NW
T_HALF = TIME // 2
T_ROW = 104


_VBLK = 4096


def _proj_body(t_ref, w_ref, b_ref, out_ref):
    out_ref[:, 0:C_PAD] = lax.dot_general(
        t_ref[...], w_ref[...],
        (((0,), (1,)), ((), ())),
        preferred_element_type=jnp.float32,
    ) + b_ref[...]


def _project_table(emb_t, fc_w_pad, fc_b_pad):
    return pl.pallas_call(
        _proj_body,
        grid=((VOCAB + _VBLK - 1) // _VBLK,),
        in_specs=[
            pl.BlockSpec((EMB, _VBLK), lambda i: (0, i)),
            pl.BlockSpec((C_PAD, EMB), lambda i: (0, 0)),
            pl.BlockSpec((1, C_PAD), lambda i: (0, 0)),
        ],
        out_specs=pl.BlockSpec((_VBLK, 128), lambda i: (i, 0)),
        out_shape=jax.ShapeDtypeStruct((VOCAB, 128), jnp.float32),
    )(emb_t, fc_w_pad, fc_b_pad)


T_CHUNK = 8
N_CHUNKS = TIME // T_CHUNK


def _sc_body(xt_hbm, proj_hbm, out_hbm, raw_v, rows_v, out_v, sem0, sem1):
    wid = lax.axis_index("s") * NC + lax.axis_index("c")
    base = wid * B_PER_W
    sems = (sem0, sem1)

    pltpu.sync_copy(xt_hbm.at[:, pl.ds(base, B_PER_W)], raw_v)

    def fire(c, buf):
        for k in range(T_CHUNK):
            pltpu.async_copy(proj_hbm.at[raw_v.at[c * T_CHUNK + k]],
                             rows_v.at[buf, k], sems[buf])

    def drain(c, buf):
        for k in range(T_CHUNK):
            pltpu.make_async_copy(proj_hbm.at[raw_v.at[c * T_CHUNK + k]],
                                  rows_v.at[buf, k], sems[buf]).wait()

    def reduce(buf, first):
        def accum(b, _):
            if first:
                a0 = rows_v[buf, 0, b, 0:16]
                a1 = rows_v[buf, 0, b, 16:32]
                k0 = 1
            else:
                a0 = out_v[b, 0:16]
                a1 = out_v[b, 16:32]
                k0 = 0
            for k in range(k0, T_CHUNK):
                a0 = a0 + rows_v[buf, k, b, 0:16]
                a1 = a1 + rows_v[buf, k, b, 16:32]
            out_v[b, 0:16] = a0
            out_v[b, 16:32] = a1
            return 0

        lax.fori_loop(0, B_PER_W, accum, 0, unroll=2)

    fire(0, 0)
    fire(1, 1)
    drain(0, 0)
    reduce(0, True)

    def step_even(g, _):
        fire(2 * g + 2, 0)
        drain(2 * g + 1, 1)
        reduce(1, False)
        fire(2 * g + 3, 1)
        drain(2 * g + 2, 0)
        reduce(0, False)
        return 0

    lax.fori_loop(0, (N_CHUNKS - 2) // 2, step_even, 0)
    fire(N_CHUNKS - 1, 0)
    drain(N_CHUNKS - 2, 1)
    reduce(1, False)
    drain(N_CHUNKS - 1, 0)
    reduce(0, False)

    pltpu.sync_copy(out_v, out_hbm.at[pl.ds(base, B_PER_W)])


@functools.cache
def _make_sc_kernel():
    return pl.kernel(
        _sc_body,
        out_type=jax.ShapeDtypeStruct((BATCH, C_PAD), jnp.float32),
        mesh=plsc.VectorSubcoreMesh(core_axis_name="c", subcore_axis_name="s"),
        scratch_types=[
            pltpu.VMEM((TIME, B_PER_W), jnp.int32),
            pltpu.VMEM((2, T_CHUNK, B_PER_W, C_PAD), jnp.float32),
            pltpu.VMEM((B_PER_W, C_PAD), jnp.float32),
            pltpu.SemaphoreType.DMA,
            pltpu.SemaphoreType.DMA,
        ],
        compiler_params=pltpu.CompilerParams(use_tc_tiling_on_sc=False,
                                             needs_layout_passes=False),
    )


def kernel(x, emb_table, fc_w, fc_b):
    fc_w_pad = jnp.zeros((C_PAD, EMB), jnp.float32).at[:NUM_CLASSES].set(fc_w)
    fc_w_pad = fc_w_pad * (1.0 / TIME)
    fc_b_pad = jnp.zeros((1, C_PAD), jnp.float32).at[0, :NUM_CLASSES].set(
        fc_b * (1.0 / TIME))
    proj = _project_table(emb_table.T, fc_w_pad, fc_b_pad)
    proj4 = proj.reshape(4 * VOCAB, C_PAD)
    out = _make_sc_kernel()(x.T * 4, proj4)
    return out[:, :NUM_CLASSES]


# speedup vs baseline: 1.1395x; 1.0230x over previous
import functools

import jax
import jax.numpy as jnp
from jax import lax
from jax.experimental import pallas as pl
from jax.experimental.pallas import tpu as pltpu
from jax.experimental.pallas import tpu_sc as plsc

VOCAB = 100000
EMB = 300
NUM_CLASSES = 20
BATCH = 4096
TIME = 200

C_PAD = 32
NC, NS = 2, 16
NW = NC * NS
B_PER_W = BATCH // NW
T_HALF = TIME // 2
T_ROW = 104


_VBLK = 8192


def _proj_body(t_ref, w_ref, b_ref, out_ref):
    out_ref[:, 0:C_PAD] = lax.dot_general(
        t_ref[...], w_ref[...],
        (((0,), (1,)), ((), ())),
        preferred_element_type=jnp.float32,
    ) + b_ref[...]


def _project_table(emb_t, fc_w_pad, fc_b_pad):
    return pl.pallas_call(
        _proj_body,
        grid=((VOCAB + _VBLK - 1) // _VBLK,),
        in_specs=[
            pl.BlockSpec((EMB, _VBLK), lambda i: (0, i)),
            pl.BlockSpec((C_PAD, EMB), lambda i: (0, 0)),
            pl.BlockSpec((1, C_PAD), lambda i: (0, 0)),
        ],
        out_specs=pl.BlockSpec((_VBLK, 128), lambda i: (i, 0)),
        out_shape=jax.ShapeDtypeStruct((VOCAB, 128), jnp.float32),
    )(emb_t, fc_w_pad, fc_b_pad)


T_CHUNK = 8
N_CHUNKS = TIME // T_CHUNK


def _sc_body(xt_hbm, proj_hbm, out_hbm, raw_v, rows_v, out_v, sem0, sem1):
    wid = lax.axis_index("s") * NC + lax.axis_index("c")
    base = wid * B_PER_W
    sems = (sem0, sem1)

    pltpu.sync_copy(xt_hbm.at[:, pl.ds(base, B_PER_W)], raw_v)

    def fire(c, buf):
        for k in range(T_CHUNK):
            pltpu.async_copy(proj_hbm.at[raw_v.at[c * T_CHUNK + k]],
                             rows_v.at[buf, k], sems[buf])

    def drain(c, buf):
        for k in range(T_CHUNK):
            pltpu.make_async_copy(proj_hbm.at[raw_v.at[c * T_CHUNK + k]],
                                  rows_v.at[buf, k], sems[buf]).wait()

    def reduce(buf, first):
        def accum(b, _):
            if first:
                a0 = rows_v[buf, 0, b, 0:16]
                a1 = rows_v[buf, 0, b, 16:32]
                k0 = 1
            else:
                a0 = out_v[b, 0:16]
                a1 = out_v[b, 16:32]
                k0 = 0
            for k in range(k0, T_CHUNK):
                a0 = a0 + rows_v[buf, k, b, 0:16]
                a1 = a1 + rows_v[buf, k, b, 16:32]
            out_v[b, 0:16] = a0
            out_v[b, 16:32] = a1
            return 0

        lax.fori_loop(0, B_PER_W, accum, 0, unroll=2)

    fire(0, 0)
    fire(1, 1)
    drain(0, 0)
    reduce(0, True)

    def step_even(g, _):
        fire(2 * g + 2, 0)
        drain(2 * g + 1, 1)
        reduce(1, False)
        fire(2 * g + 3, 1)
        drain(2 * g + 2, 0)
        reduce(0, False)
        return 0

    lax.fori_loop(0, (N_CHUNKS - 2) // 2, step_even, 0)
    fire(N_CHUNKS - 1, 0)
    drain(N_CHUNKS - 2, 1)
    reduce(1, False)
    drain(N_CHUNKS - 1, 0)
    reduce(0, False)

    pltpu.sync_copy(out_v, out_hbm.at[pl.ds(base, B_PER_W)])


@functools.cache
def _make_sc_kernel():
    return pl.kernel(
        _sc_body,
        out_type=jax.ShapeDtypeStruct((BATCH, C_PAD), jnp.float32),
        mesh=plsc.VectorSubcoreMesh(core_axis_name="c", subcore_axis_name="s"),
        scratch_types=[
            pltpu.VMEM((TIME, B_PER_W), jnp.int32),
            pltpu.VMEM((2, T_CHUNK, B_PER_W, C_PAD), jnp.float32),
            pltpu.VMEM((B_PER_W, C_PAD), jnp.float32),
            pltpu.SemaphoreType.DMA,
            pltpu.SemaphoreType.DMA,
        ],
        compiler_params=pltpu.CompilerParams(use_tc_tiling_on_sc=False,
                                             needs_layout_passes=False),
    )


def kernel(x, emb_table, fc_w, fc_b):
    fc_w_pad = jnp.zeros((C_PAD, EMB), jnp.float32).at[:NUM_CLASSES].set(fc_w)
    fc_w_pad = fc_w_pad * (1.0 / TIME)
    fc_b_pad = jnp.zeros((1, C_PAD), jnp.float32).at[0, :NUM_CLASSES].set(
        fc_b * (1.0 / TIME))
    proj = _project_table(emb_table.T, fc_w_pad, fc_b_pad)
    proj4 = proj.reshape(4 * VOCAB, C_PAD)
    out = _make_sc_kernel()(x.T * 4, proj4)
    return out[:, :NUM_CLASSES]
